# baseline (speedup 1.0000x reference)
import functools

import jax
import jax.numpy as jnp
from jax import lax
from jax.experimental import pallas as pl
from jax.experimental.pallas import tpu as pltpu
from jax.experimental.pallas import tpu_sc as plsc

N = 10000
D_IN = 256
D_OUT = 256
HALF = 128
NT = 16
CHUNK = 128
CH = 80
CH2 = CH // 2
E_PAD = NT * CH * CHUNK
ROWS_PER_TILE = 624


def _mm_body(x_ref, w_ref, out_ref):
    y = jnp.dot(x_ref[...], w_ref[...], preferred_element_type=jnp.float32)
    out_ref[...] = jnp.where(y >= 0.0, y, 0.2 * y)


def _support_blocked(features, W):
    bn = 1000
    grid = (N // bn,)
    return pl.pallas_call(
        _mm_body,
        grid=grid,
        in_specs=[
            pl.BlockSpec((bn, D_IN), lambda i: (i, 0)),
            pl.BlockSpec((D_IN, D_OUT), lambda i: (0, 0)),
        ],
        out_specs=pl.BlockSpec((bn, D_OUT), lambda i: (i, 0)),
        out_shape=jax.ShapeDtypeStruct((N, D_OUT), jnp.float32),
    )(features, W)


def _bcast_lane(vec, lane):
    idx = jnp.full((16, 1), lane, jnp.int32)
    return lax.gather(
        vec,
        idx,
        lax.GatherDimensionNumbers(
            offset_dims=(), collapsed_slice_dims=(0,), start_index_map=(0,)
        ),
        (1,),
        mode=lax.GatherScatterMode.PROMISE_IN_BOUNDS,
    )

def _sc_body(
    sup, eis, ws, out,
    src_v, dst_v, w_v, rows_a, rows_b, acc, gsem, ssem,
):
    srcs = eis.at[1]
    dsts = eis.at[0]
    c = lax.axis_index("c")
    s = lax.axis_index("s")
    rows_v = rows_a
    sup_c = sup.at[:, pl.ds(c * HALF, HALF)]

    pltpu.sync_copy(srcs.at[s].at[pl.ds(0, CH2)], src_v)
    pltpu.sync_copy(dsts.at[s].at[pl.ds(0, CH2)], dst_v)
    pltpu.sync_copy(ws.at[s].at[pl.ds(0, CH2 * CHUNK)], w_v)

    z16 = jnp.zeros((16,), jnp.float32)

    def zrow(i, carry):
        for cb in range(8):
            rows_v[i, pl.ds(cb * 16, 16)] = z16
        return carry

    lax.fori_loop(0, CHUNK, zrow, 0)
    for k in range(6):
        pltpu.sync_copy(
            rows_v.at[pl.ds(0, 104)],
            acc.at[pl.ds(s * ROWS_PER_TILE + k * 104, 104)],
        )

    @pl.when(s == 0)
    def _zero_tail():
        pltpu.sync_copy(rows_v.at[pl.ds(0, 16)], acc.at[pl.ds(9984, 16)])

    plsc.subcore_barrier()

    rows = (rows_a, rows_b)

    def scale_rows(buf, jr):
        def group_body(g, gcarry):
            w16 = w_v[pl.ds(jr * CHUNK + g * 16, 16)]
            for l in range(16):
                wvec = _bcast_lane(w16, l)
                e = g * 16 + l
                for cb in range(8):
                    buf[e, pl.ds(cb * 16, 16)] = (
                        buf[e, pl.ds(cb * 16, 16)] * wvec
                    )
            return gcarry

        lax.fori_loop(0, CHUNK // 16, group_body, 0)

    def start_gather(jrow, buf):
        pltpu.async_copy(sup_c.at[src_v.at[jrow]], buf, gsem)

    def wait_gather(jrow, buf):
        pltpu.make_async_copy(sup_c.at[src_v.at[jrow]], buf, gsem).wait()

    start_gather(0, rows_a)

    def j2_body(j2, carry):
        for b in range(2):
            j = j2 * 2 + b
            jr = lax.rem(j, CH2)
            jr1 = lax.rem(j + 1, CH2)
            cur = rows[b]
            oth = rows[1 - b]
            wait_gather(jr, cur)
            if b == 0:
                @pl.when(j2 > 0)
                def _wait_prev_scatter():
                    pltpu.make_async_copy(
                        oth, acc.at[dst_v.at[jr]], ssem
                    ).wait()

                @pl.when(j2 == CH2 // 2)
                def _restage_dst_w():
                    pltpu.sync_copy(dsts.at[s].at[pl.ds(CH2, CH2)], dst_v)
                    pltpu.sync_copy(
                        ws.at[s].at[pl.ds(CH2 * CHUNK, CH2 * CHUNK)], w_v
                    )

                start_gather(jr1, oth)
            else:
                pltpu.make_async_copy(oth, acc.at[dst_v.at[jr]], ssem).wait()

                @pl.when(j2 == CH2 // 2 - 1)
                def _restage_src():
                    pltpu.sync_copy(srcs.at[s].at[pl.ds(CH2, CH2)], src_v)

                @pl.when(j2 < CH // 2 - 1)
                def _next_gather():
                    start_gather(jr1, oth)

            scale_rows(cur, jr)
            pltpu.async_copy(cur, acc.at[dst_v.at[jr]], ssem, add=True)
        return carry

    lax.fori_loop(0, CH // 2, j2_body, 0)
    pltpu.make_async_copy(rows_b, acc.at[dst_v.at[CH2 - 1]], ssem).wait()
    plsc.subcore_barrier()

    pltpu.sync_copy(
        acc.at[pl.ds(s * ROWS_PER_TILE, ROWS_PER_TILE)],
        out.at[pl.ds(s * ROWS_PER_TILE, ROWS_PER_TILE), pl.ds(c * HALF, HALF)],
    )

    @pl.when(s == 0)
    def _write_tail():
        pltpu.sync_copy(
            acc.at[pl.ds(9984, 16)],
            out.at[pl.ds(9984, 16), pl.ds(c * HALF, HALF)],
        )


def _spmm(sup_blocked, ei_p, w_p):
    mesh = plsc.VectorSubcoreMesh(core_axis_name="c", subcore_axis_name="s")
    f = pl.kernel(
        _sc_body,
        out_type=jax.ShapeDtypeStruct((N, D_OUT), jnp.float32),
        mesh=mesh,
        scratch_types=[
            pltpu.VMEM((CH2, CHUNK), jnp.int32),
            pltpu.VMEM((CH2, CHUNK), jnp.int32),
            pltpu.VMEM((CH2 * CHUNK,), jnp.float32),
            pltpu.VMEM((CHUNK, HALF), jnp.float32),
            pltpu.VMEM((CHUNK, HALF), jnp.float32),
            pltpu.VMEM_SHARED((N, HALF), jnp.float32),
            pltpu.SemaphoreType.DMA,
            pltpu.SemaphoreType.DMA,
        ],
    )
    return f(sup_blocked, ei_p, w_p)


@jax.jit
def kernel(features, edge_index, edge_weight, W):
    sup = _support_blocked(features, W)

    e = edge_index.shape[1]
    pad = E_PAD - e
    spread = (jnp.arange(pad, dtype=jnp.int32) * 61) % N
    ei_p = jnp.concatenate(
        [edge_index.astype(jnp.int32), jnp.tile(spread, (2, 1))], axis=1
    ).reshape(2, NT, CH, CHUNK)
    w_p = jnp.pad(edge_weight, (0, pad)).reshape(NT, CH * CHUNK)

    return _spmm(sup, ei_p, w_p)


# speedup vs baseline: 1.0168x; 1.0168x over previous
import functools

import jax
import jax.numpy as jnp
from jax import lax
from jax.experimental import pallas as pl
from jax.experimental.pallas import tpu as pltpu
from jax.experimental.pallas import tpu_sc as plsc

N = 10000
D_IN = 256
D_OUT = 256
HALF = 128
NT = 16
CHUNK = 128
CH = 80
CH2 = CH // 2
E_PAD = NT * CH * CHUNK
ROWS_PER_TILE = 624


def _mm_body(x_ref, w_ref, out_ref):
    y = jnp.dot(x_ref[...], w_ref[...], preferred_element_type=jnp.float32)
    out_ref[...] = jnp.where(y >= 0.0, y, 0.2 * y)


def _support_blocked(features, W):
    bn = 2000
    grid = (N // bn,)
    return pl.pallas_call(
        _mm_body,
        grid=grid,
        in_specs=[
            pl.BlockSpec((bn, D_IN), lambda i: (i, 0)),
            pl.BlockSpec((D_IN, D_OUT), lambda i: (0, 0)),
        ],
        out_specs=pl.BlockSpec((bn, D_OUT), lambda i: (i, 0)),
        out_shape=jax.ShapeDtypeStruct((N, D_OUT), jnp.float32),
    )(features, W)


def _bcast_lane(vec, lane):
    idx = jnp.full((16, 1), lane, jnp.int32)
    return lax.gather(
        vec,
        idx,
        lax.GatherDimensionNumbers(
            offset_dims=(), collapsed_slice_dims=(0,), start_index_map=(0,)
        ),
        (1,),
        mode=lax.GatherScatterMode.PROMISE_IN_BOUNDS,
    )

def _sc_body(
    sup, eis, ws, out,
    src_v, dst_v, w_v, rows_a, rows_b, acc, gsem, ssem,
):
    srcs = eis.at[1]
    dsts = eis.at[0]
    c = lax.axis_index("c")
    s = lax.axis_index("s")
    rows_v = rows_a
    sup_c = sup.at[:, pl.ds(c * HALF, HALF)]

    pltpu.sync_copy(srcs.at[s].at[pl.ds(0, CH2)], src_v)
    pltpu.sync_copy(dsts.at[s].at[pl.ds(0, CH2)], dst_v)
    pltpu.sync_copy(ws.at[s].at[pl.ds(0, CH2 * CHUNK)], w_v)

    z16 = jnp.zeros((16,), jnp.float32)

    def zrow(i, carry):
        for cb in range(8):
            rows_v[i, pl.ds(cb * 16, 16)] = z16
        return carry

    lax.fori_loop(0, CHUNK, zrow, 0)
    for k in range(6):
        pltpu.sync_copy(
            rows_v.at[pl.ds(0, 104)],
            acc.at[pl.ds(s * ROWS_PER_TILE + k * 104, 104)],
        )

    @pl.when(s == 0)
    def _zero_tail():
        pltpu.sync_copy(rows_v.at[pl.ds(0, 16)], acc.at[pl.ds(9984, 16)])

    plsc.subcore_barrier()

    rows = (rows_a, rows_b)

    def scale_rows(buf, jr):
        def group_body(g, gcarry):
            w16 = w_v[pl.ds(jr * CHUNK + g * 16, 16)]
            for l in range(16):
                wvec = _bcast_lane(w16, l)
                e = g * 16 + l
                for cb in range(8):
                    buf[e, pl.ds(cb * 16, 16)] = (
                        buf[e, pl.ds(cb * 16, 16)] * wvec
                    )
            return gcarry

        lax.fori_loop(0, CHUNK // 16, group_body, 0)

    def start_gather(jrow, buf):
        pltpu.async_copy(sup_c.at[src_v.at[jrow]], buf, gsem)

    def wait_gather(jrow, buf):
        pltpu.make_async_copy(sup_c.at[src_v.at[jrow]], buf, gsem).wait()

    start_gather(0, rows_a)

    def j2_body(j2, carry):
        for b in range(2):
            j = j2 * 2 + b
            jr = lax.rem(j, CH2)
            jr1 = lax.rem(j + 1, CH2)
            cur = rows[b]
            oth = rows[1 - b]
            wait_gather(jr, cur)
            if b == 0:
                @pl.when(j2 > 0)
                def _wait_prev_scatter():
                    pltpu.make_async_copy(
                        oth, acc.at[dst_v.at[jr]], ssem
                    ).wait()

                @pl.when(j2 == CH2 // 2)
                def _restage_dst_w():
                    pltpu.sync_copy(dsts.at[s].at[pl.ds(CH2, CH2)], dst_v)
                    pltpu.sync_copy(
                        ws.at[s].at[pl.ds(CH2 * CHUNK, CH2 * CHUNK)], w_v
                    )

                start_gather(jr1, oth)
            else:
                pltpu.make_async_copy(oth, acc.at[dst_v.at[jr]], ssem).wait()

                @pl.when(j2 == CH2 // 2 - 1)
                def _restage_src():
                    pltpu.sync_copy(srcs.at[s].at[pl.ds(CH2, CH2)], src_v)

                @pl.when(j2 < CH // 2 - 1)
                def _next_gather():
                    start_gather(jr1, oth)

            scale_rows(cur, jr)
            pltpu.async_copy(cur, acc.at[dst_v.at[jr]], ssem, add=True)
        return carry

    lax.fori_loop(0, CH // 2, j2_body, 0)
    pltpu.make_async_copy(rows_b, acc.at[dst_v.at[CH2 - 1]], ssem).wait()
    plsc.subcore_barrier()

    pltpu.sync_copy(
        acc.at[pl.ds(s * ROWS_PER_TILE, ROWS_PER_TILE)],
        out.at[pl.ds(s * ROWS_PER_TILE, ROWS_PER_TILE), pl.ds(c * HALF, HALF)],
    )

    @pl.when(s == 0)
    def _write_tail():
        pltpu.sync_copy(
            acc.at[pl.ds(9984, 16)],
            out.at[pl.ds(9984, 16), pl.ds(c * HALF, HALF)],
        )


def _spmm(sup_blocked, ei_p, w_p):
    mesh = plsc.VectorSubcoreMesh(core_axis_name="c", subcore_axis_name="s")
    f = pl.kernel(
        _sc_body,
        out_type=jax.ShapeDtypeStruct((N, D_OUT), jnp.float32),
        mesh=mesh,
        scratch_types=[
            pltpu.VMEM((CH2, CHUNK), jnp.int32),
            pltpu.VMEM((CH2, CHUNK), jnp.int32),
            pltpu.VMEM((CH2 * CHUNK,), jnp.float32),
            pltpu.VMEM((CHUNK, HALF), jnp.float32),
            pltpu.VMEM((CHUNK, HALF), jnp.float32),
            pltpu.VMEM_SHARED((N, HALF), jnp.float32),
            pltpu.SemaphoreType.DMA,
            pltpu.SemaphoreType.DMA,
        ],
    )
    return f(sup_blocked, ei_p, w_p)


@jax.jit
def kernel(features, edge_index, edge_weight, W):
    sup = _support_blocked(features, W)

    e = edge_index.shape[1]
    pad = E_PAD - e
    spread = (jnp.arange(pad, dtype=jnp.int32) * 61) % N
    ei_p = jnp.concatenate(
        [edge_index.astype(jnp.int32), jnp.tile(spread, (2, 1))], axis=1
    ).reshape(2, NT, CH, CHUNK)
    w_p = jnp.pad(edge_weight, (0, pad)).reshape(NT, CH * CHUNK)

    return _spmm(sup, ei_p, w_p)
